# baseline (speedup 1.0000x reference)
import functools

import jax
import jax.numpy as jnp
from jax import lax
from jax.experimental import pallas as pl
from jax.experimental.pallas import tpu as pltpu
from jax.experimental.pallas import tpu_sc as plsc

N = 10000
E = 320000
H = 128

NC = 2
NS = 16
NT = NC * NS
CH = 128
NCHUNK = 80
NPHASE = 2
PCH = NCHUNK // NPHASE
ET = CH * NCHUNK
EP = NT * ET
NPAD = N + 8
ZR = 624
ZTAIL = N - NS * ZR

_mesh = plsc.VectorSubcoreMesh(
    core_axis_name="c", subcore_axis_name="s", num_cores=NC, num_subcores=NS
)


@functools.partial(
    pl.kernel,
    out_type=jax.ShapeDtypeStruct((NC, N, H), jnp.float32),
    mesh=_mesh,
    scratch_types=[
        pltpu.VMEM((PCH, CH), jnp.int32),
        pltpu.VMEM((PCH, CH), jnp.int32),
        pltpu.VMEM((CH, H), jnp.float32),
        pltpu.VMEM((CH, H), jnp.float32),
        pltpu.VMEM_SHARED((NPAD, H), jnp.float32),
        pltpu.SemaphoreType.DMA,
        pltpu.SemaphoreType.DMA,
        pltpu.SemaphoreType.DMA,
        pltpu.SemaphoreType.DMA,
    ],
)
def _segsum(tab, src2d, dst2d, zeros, out,
            src_all, dst_all, rows0, rows1, acc, sg0, sg1, ss0, ss1):
    cid = lax.axis_index("c")
    sid = lax.axis_index("s")
    tid = cid * NS + sid
    rows_v = (rows0, rows1)
    sem_g = (sg0, sg1)
    sem_s = (ss0, ss1)

    r0 = pl.multiple_of(sid * ZR, 8)
    pltpu.async_copy(zeros.at[pl.ds(0, ZR)], acc.at[pl.ds(r0, ZR)], sem_s[0])

    @pl.when(sid == 0)
    def _():
        pltpu.async_copy(zeros.at[pl.ds(0, NPAD - NS * ZR)],
                         acc.at[pl.ds(NS * ZR, NPAD - NS * ZR)], sem_s[1])

    for phase in range(NPHASE):
        ib = pl.multiple_of(tid * NCHUNK + phase * PCH, 8)
        pltpu.sync_copy(src2d.at[pl.ds(ib, PCH)], src_all)
        pltpu.sync_copy(dst2d.at[pl.ds(ib, PCH)], dst_all)

        pltpu.async_copy(tab.at[src_all.at[0]], rows_v[0], sem_g[0])

        if phase == 0:
            pltpu.make_async_copy(
                zeros.at[pl.ds(0, ZR)], acc.at[pl.ds(r0, ZR)],
                sem_s[0]).wait()

            @pl.when(sid == 0)
            def _():
                pltpu.make_async_copy(
                    zeros.at[pl.ds(0, NPAD - NS * ZR)],
                    acc.at[pl.ds(NS * ZR, NPAD - NS * ZR)], sem_s[1]).wait()

            plsc.subcore_barrier()

        def pair(j2, carry):
            for b in (0, 1):
                j = j2 * 2 + b
                nb = 1 - b

                @pl.when(j >= 1)
                def _():
                    pltpu.make_async_copy(
                        rows_v[nb], acc.at[dst_all.at[j - 1]],
                        sem_s[nb]).wait()

                @pl.when(j + 1 < PCH)
                def _():
                    pltpu.async_copy(
                        tab.at[src_all.at[j + 1]], rows_v[nb], sem_g[nb])

                pltpu.make_async_copy(
                    tab.at[src_all.at[j]], rows_v[b], sem_g[b]).wait()
                pltpu.async_copy(rows_v[b], acc.at[dst_all.at[j]], sem_s[b],
                                 add=True)
            return carry

        lax.fori_loop(0, PCH // 2, pair, 0)
        pltpu.make_async_copy(
            rows_v[1], acc.at[dst_all.at[PCH - 1]], sem_s[1]).wait()
    plsc.subcore_barrier()
    pltpu.sync_copy(acc.at[pl.ds(r0, ZR)], out.at[cid, pl.ds(r0, ZR)])

    @pl.when(sid == 0)
    def _():
        pltpu.sync_copy(acc.at[pl.ds(NS * ZR, ZTAIL)],
                        out.at[cid, pl.ds(NS * ZR, ZTAIL)])


BN = 1000


def _mlp(z, W1, b1, W2, b2):
    t = jnp.maximum(
        jnp.dot(z, W1, preferred_element_type=jnp.float32) + b1, 0.0
    )
    return jnp.dot(t, W2, preferred_element_type=jnp.float32) + b2


def _tc1_body(x_ref, a_ref, W11, b11, W21, b21,
              W12, b12, W22, b22, h2a_ref, h3a_ref):
    z = x_ref[...] + a_ref[0] + a_ref[1]
    h2a_ref[...] = _mlp(z, W11[...], b11[...], W21[...], b21[...])
    h3a_ref[...] = _mlp(z, W12[...], b12[...], W22[...], b22[...])


def _tc1b_body(x_ref, a_ref, W10, b10, W20, b20, out1_ref):
    z = x_ref[...] + a_ref[0] + a_ref[1]
    out1_ref[...] = _mlp(z, W10[...], b10[...], W20[...], b20[...])


def _tc2_body(h2a_ref, a2_ref, h3a_ref, a3_ref, W11, b11, W21, b21,
              W12, b12, W22, b22, out2_ref, h3b_ref):
    z2 = h2a_ref[...] + a2_ref[0] + a2_ref[1]
    out2_ref[...] = _mlp(z2, W11[...], b11[...], W21[...], b21[...])
    z3 = h3a_ref[...] + a3_ref[0] + a3_ref[1]
    h3b_ref[...] = _mlp(z3, W12[...], b12[...], W22[...], b22[...])


def _tc3_body(h3b_ref, a4_ref, out1_ref, out2_ref, W12, b12, W22, b22,
              Wp_ref, bp_ref, y_ref):
    z = h3b_ref[...] + a4_ref[0] + a4_ref[1]
    out3 = _mlp(z, W12[...], b12[...], W22[...], b22[...])
    Wp = Wp_ref[...]
    y = jnp.dot(out1_ref[...], Wp[0:H], preferred_element_type=jnp.float32)
    y += jnp.dot(out2_ref[...], Wp[H:2 * H], preferred_element_type=jnp.float32)
    y += jnp.dot(out3, Wp[2 * H:3 * H], preferred_element_type=jnp.float32)
    y_ref[...] = y + bp_ref[...]


_row_spec = pl.BlockSpec((BN, H), lambda i: (i, 0))
_agg_spec = pl.BlockSpec((NC, BN, H), lambda i: (0, i, 0))
_w_spec = pl.BlockSpec((H, H), lambda i: (0, 0))
_b_spec = pl.BlockSpec((1, H), lambda i: (0, 0))
_out_nh = jax.ShapeDtypeStruct((N, H), jnp.float32)
_grid = (N // BN,)
_tc_params = pltpu.CompilerParams(dimension_semantics=("arbitrary",))

_tc1 = pl.pallas_call(
    _tc1_body,
    grid=_grid,
    in_specs=[_row_spec, _agg_spec] + [_w_spec, _b_spec] * 4,
    out_specs=[_row_spec] * 2,
    out_shape=[_out_nh] * 2,
    compiler_params=_tc_params,
)

_tc1b = pl.pallas_call(
    _tc1b_body,
    grid=_grid,
    in_specs=[_row_spec, _agg_spec] + [_w_spec, _b_spec] * 2,
    out_specs=_row_spec,
    out_shape=_out_nh,
    compiler_params=_tc_params,
)

_tc2 = pl.pallas_call(
    _tc2_body,
    grid=_grid,
    in_specs=[_row_spec, _agg_spec, _row_spec, _agg_spec]
    + [_w_spec, _b_spec] * 4,
    out_specs=[_row_spec] * 2,
    out_shape=[_out_nh] * 2,
    compiler_params=_tc_params,
)

_tc3 = pl.pallas_call(
    _tc3_body,
    grid=_grid,
    in_specs=[_row_spec, _agg_spec, _row_spec, _row_spec]
    + [_w_spec, _b_spec] * 2
    + [pl.BlockSpec((3 * H, H), lambda i: (0, 0)), _b_spec],
    out_specs=_row_spec,
    out_shape=_out_nh,
    compiler_params=_tc_params,
)


def kernel(x, edge_index, W1_0, b1_0, W2_0, b2_0, W1_1, b1_1, W2_1, b2_1,
           W1_2, b1_2, W2_2, b2_2, Wp, bp):
    pad = EP - E
    src = jnp.concatenate(
        [edge_index[0], jnp.arange(pad, dtype=edge_index.dtype) % N]
    ).reshape(EP // CH, CH)
    dst = jnp.concatenate(
        [edge_index[1],
         N + (jnp.arange(pad, dtype=edge_index.dtype) % 8)]
    ).reshape(EP // CH, CH)
    zeros = jnp.zeros((ZR, H), jnp.float32)
    b1_0r, b2_0r = b1_0.reshape(1, H), b2_0.reshape(1, H)
    b1_1r, b2_1r = b1_1.reshape(1, H), b2_1.reshape(1, H)
    b1_2r, b2_2r = b1_2.reshape(1, H), b2_2.reshape(1, H)
    bpr = bp.reshape(1, H)

    agg0 = _segsum(x, src, dst, zeros)
    h2a, h3a = _tc1(x, agg0, W1_1, b1_1r, W2_1, b2_1r,
                    W1_2, b1_2r, W2_2, b2_2r)
    agg2 = _segsum(h2a, src, dst, zeros)
    out1 = _tc1b(x, agg0, W1_0, b1_0r, W2_0, b2_0r)
    agg3a = _segsum(h3a, src, dst, zeros)
    out2, h3b = _tc2(h2a, agg2, h3a, agg3a,
                     W1_1, b1_1r, W2_1, b2_1r, W1_2, b1_2r, W2_2, b2_2r)
    agg3b = _segsum(h3b, src, dst, zeros)
    y = _tc3(h3b, agg3b, out1, out2, W1_2, b1_2r, W2_2, b2_2r, Wp, bpr)
    return y


# speedup vs baseline: 1.0477x; 1.0477x over previous
import functools

import jax
import jax.numpy as jnp
from jax import lax
from jax.experimental import pallas as pl
from jax.experimental.pallas import tpu as pltpu
from jax.experimental.pallas import tpu_sc as plsc

N = 10000
E = 320000
H = 128

NC = 2
NS = 16
NT = NC * NS
CH = 128
NCHUNK = 80
NPHASE = 2
PCH = NCHUNK // NPHASE
ET = CH * NCHUNK
EP = NT * ET
NPAD = N + 8
ZR = 624
ZTAIL = N - NS * ZR

_mesh = plsc.VectorSubcoreMesh(
    core_axis_name="c", subcore_axis_name="s", num_cores=NC, num_subcores=NS
)


def _make_segsum(two_tables):
    nchunk = NCHUNK * NC if two_tables else NCHUNK
    nphase = NPHASE * NC if two_tables else NPHASE

    @functools.partial(
        pl.kernel,
        out_type=jax.ShapeDtypeStruct((NC, N, H), jnp.float32),
        mesh=_mesh,
        scratch_types=[
            pltpu.VMEM((PCH, CH), jnp.int32),
            pltpu.VMEM((PCH, CH), jnp.int32),
            pltpu.VMEM((CH, H), jnp.float32),
            pltpu.VMEM((CH, H), jnp.float32),
            pltpu.VMEM_SHARED((NPAD, H), jnp.float32),
            pltpu.SemaphoreType.DMA,
            pltpu.SemaphoreType.DMA,
            pltpu.SemaphoreType.DMA,
            pltpu.SemaphoreType.DMA,
        ],
    )
    def segsum(tabs, src2d, dst2d, zeros, out,
               src_all, dst_all, rows0, rows1, acc, sg0, sg1, ss0, ss1):
        cid = lax.axis_index("c")
        sid = lax.axis_index("s")
        rows_v = (rows0, rows1)
        sem_g = (sg0, sg1)
        sem_s = (ss0, ss1)
        if two_tables:
            tab = tabs.at[cid]
            cb = sid * nchunk
        else:
            tab = tabs
            cb = (cid * NS + sid) * nchunk

        r0 = pl.multiple_of(sid * ZR, 8)
        pltpu.async_copy(zeros.at[pl.ds(0, ZR)], acc.at[pl.ds(r0, ZR)],
                         sem_s[0])

        @pl.when(sid == 0)
        def _():
            pltpu.async_copy(zeros.at[pl.ds(0, NPAD - NS * ZR)],
                             acc.at[pl.ds(NS * ZR, NPAD - NS * ZR)], sem_s[1])

        for phase in range(nphase):
            ib = pl.multiple_of(cb + phase * PCH, 8)
            pltpu.sync_copy(src2d.at[pl.ds(ib, PCH)], src_all)
            pltpu.sync_copy(dst2d.at[pl.ds(ib, PCH)], dst_all)

            pltpu.async_copy(tab.at[src_all.at[0]], rows_v[0], sem_g[0])

            if phase == 0:
                pltpu.make_async_copy(
                    zeros.at[pl.ds(0, ZR)], acc.at[pl.ds(r0, ZR)],
                    sem_s[0]).wait()

                @pl.when(sid == 0)
                def _():
                    pltpu.make_async_copy(
                        zeros.at[pl.ds(0, NPAD - NS * ZR)],
                        acc.at[pl.ds(NS * ZR, NPAD - NS * ZR)],
                        sem_s[1]).wait()

                plsc.subcore_barrier()

            def pair(j2, carry):
                for b in (0, 1):
                    j = j2 * 2 + b
                    nb = 1 - b

                    @pl.when(j >= 1)
                    def _():
                        pltpu.make_async_copy(
                            rows_v[nb], acc.at[dst_all.at[j - 1]],
                            sem_s[nb]).wait()

                    @pl.when(j + 1 < PCH)
                    def _():
                        pltpu.async_copy(
                            tab.at[src_all.at[j + 1]], rows_v[nb], sem_g[nb])

                    pltpu.make_async_copy(
                        tab.at[src_all.at[j]], rows_v[b], sem_g[b]).wait()
                    pltpu.async_copy(rows_v[b], acc.at[dst_all.at[j]],
                                     sem_s[b], add=True)
                return carry

            lax.fori_loop(0, PCH // 2, pair, 0)
            pltpu.make_async_copy(
                rows_v[1], acc.at[dst_all.at[PCH - 1]], sem_s[1]).wait()
        plsc.subcore_barrier()
        pltpu.sync_copy(acc.at[pl.ds(r0, ZR)], out.at[cid, pl.ds(r0, ZR)])

        @pl.when(sid == 0)
        def _():
            pltpu.sync_copy(acc.at[pl.ds(NS * ZR, ZTAIL)],
                            out.at[cid, pl.ds(NS * ZR, ZTAIL)])

    return segsum


_segsum = _make_segsum(False)
_segsum2 = _make_segsum(True)


BN = 1000


def _mlp(z, W1, b1, W2, b2):
    t = jnp.maximum(
        jnp.dot(z, W1, preferred_element_type=jnp.float32) + b1, 0.0
    )
    return jnp.dot(t, W2, preferred_element_type=jnp.float32) + b2


def _tc1_body(x_ref, a_ref, W10, b10, W20, b20, W11, b11, W21, b21,
              W12, b12, W22, b22, out1_ref, h23_ref):
    z = x_ref[...] + a_ref[0] + a_ref[1]
    out1_ref[...] = _mlp(z, W10[...], b10[...], W20[...], b20[...])
    h23_ref[0] = _mlp(z, W11[...], b11[...], W21[...], b21[...])
    h23_ref[1] = _mlp(z, W12[...], b12[...], W22[...], b22[...])


def _tc2_body(h23_ref, a23_ref, W11, b11, W21, b21,
              W12, b12, W22, b22, out2_ref, h3b_ref):
    z2 = h23_ref[0] + a23_ref[0]
    out2_ref[...] = _mlp(z2, W11[...], b11[...], W21[...], b21[...])
    z3 = h23_ref[1] + a23_ref[1]
    h3b_ref[...] = _mlp(z3, W12[...], b12[...], W22[...], b22[...])


def _tc3_body(h3b_ref, a4_ref, out1_ref, out2_ref, W12, b12, W22, b22,
              Wp_ref, bp_ref, y_ref):
    z = h3b_ref[...] + a4_ref[0] + a4_ref[1]
    out3 = _mlp(z, W12[...], b12[...], W22[...], b22[...])
    Wp = Wp_ref[...]
    y = jnp.dot(out1_ref[...], Wp[0:H], preferred_element_type=jnp.float32)
    y += jnp.dot(out2_ref[...], Wp[H:2 * H], preferred_element_type=jnp.float32)
    y += jnp.dot(out3, Wp[2 * H:3 * H], preferred_element_type=jnp.float32)
    y_ref[...] = y + bp_ref[...]


_row_spec = pl.BlockSpec((BN, H), lambda i: (i, 0))
_agg_spec = pl.BlockSpec((NC, BN, H), lambda i: (0, i, 0))
_w_spec = pl.BlockSpec((H, H), lambda i: (0, 0))
_b_spec = pl.BlockSpec((1, H), lambda i: (0, 0))
_out_nh = jax.ShapeDtypeStruct((N, H), jnp.float32)
_grid = (N // BN,)
_tc_params = pltpu.CompilerParams(dimension_semantics=("arbitrary",))

_tc1 = pl.pallas_call(
    _tc1_body,
    grid=_grid,
    in_specs=[_row_spec, _agg_spec] + [_w_spec, _b_spec] * 6,
    out_specs=[_row_spec, _agg_spec],
    out_shape=[_out_nh, jax.ShapeDtypeStruct((NC, N, H), jnp.float32)],
    compiler_params=_tc_params,
)

_tc2 = pl.pallas_call(
    _tc2_body,
    grid=_grid,
    in_specs=[_agg_spec, _agg_spec] + [_w_spec, _b_spec] * 4,
    out_specs=[_row_spec] * 2,
    out_shape=[_out_nh] * 2,
    compiler_params=_tc_params,
)

_tc3 = pl.pallas_call(
    _tc3_body,
    grid=_grid,
    in_specs=[_row_spec, _agg_spec, _row_spec, _row_spec]
    + [_w_spec, _b_spec] * 2
    + [pl.BlockSpec((3 * H, H), lambda i: (0, 0)), _b_spec],
    out_specs=_row_spec,
    out_shape=_out_nh,
    compiler_params=_tc_params,
)


def kernel(x, edge_index, W1_0, b1_0, W2_0, b2_0, W1_1, b1_1, W2_1, b2_1,
           W1_2, b1_2, W2_2, b2_2, Wp, bp):
    pad = EP - E
    src = jnp.concatenate(
        [edge_index[0], jnp.arange(pad, dtype=edge_index.dtype) % N]
    ).reshape(EP // CH, CH)
    dst = jnp.concatenate(
        [edge_index[1],
         N + (jnp.arange(pad, dtype=edge_index.dtype) % 8)]
    ).reshape(EP // CH, CH)
    zeros = jnp.zeros((ZR, H), jnp.float32)
    b1_0r, b2_0r = b1_0.reshape(1, H), b2_0.reshape(1, H)
    b1_1r, b2_1r = b1_1.reshape(1, H), b2_1.reshape(1, H)
    b1_2r, b2_2r = b1_2.reshape(1, H), b2_2.reshape(1, H)
    bpr = bp.reshape(1, H)

    agg0 = _segsum(x, src, dst, zeros)
    out1, h23 = _tc1(x, agg0, W1_0, b1_0r, W2_0, b2_0r,
                     W1_1, b1_1r, W2_1, b2_1r, W1_2, b1_2r, W2_2, b2_2r)
    agg23 = _segsum2(h23, src, dst, zeros)
    out2, h3b = _tc2(h23, agg23,
                     W1_1, b1_1r, W2_1, b2_1r, W1_2, b1_2r, W2_2, b2_2r)
    agg3b = _segsum(h3b, src, dst, zeros)
    y = _tc3(h3b, agg3b, out1, out2, W1_2, b1_2r, W2_2, b2_2r, Wp, bpr)
    return y


# speedup vs baseline: 1.0695x; 1.0207x over previous
import functools

import jax
import jax.numpy as jnp
from jax import lax
from jax.experimental import pallas as pl
from jax.experimental.pallas import tpu as pltpu
from jax.experimental.pallas import tpu_sc as plsc

N = 10000
E = 320000
H = 128

NC = 2
NS = 16
NT = NC * NS
CH = 128
NCHUNK = 80
NPHASE = 2
PCH = NCHUNK // NPHASE
ET = CH * NCHUNK
EP = NT * ET
NPAD = N + 8
ZR = 624
ZTAIL = N - NS * ZR

_mesh = plsc.VectorSubcoreMesh(
    core_axis_name="c", subcore_axis_name="s", num_cores=NC, num_subcores=NS
)


def _make_segsum(two_tables):
    nchunk = NCHUNK * NC if two_tables else NCHUNK
    nphase = NPHASE * NC if two_tables else NPHASE

    @functools.partial(
        pl.kernel,
        out_type=jax.ShapeDtypeStruct((NC, N, H), jnp.float32),
        mesh=_mesh,
        scratch_types=[
            pltpu.VMEM((PCH, CH), jnp.int32),
            pltpu.VMEM((PCH, CH), jnp.int32),
            pltpu.VMEM((CH, H), jnp.float32),
            pltpu.VMEM((CH, H), jnp.float32),
            pltpu.VMEM_SHARED((NPAD, H), jnp.float32),
            pltpu.SemaphoreType.DMA,
            pltpu.SemaphoreType.DMA,
            pltpu.SemaphoreType.DMA,
            pltpu.SemaphoreType.DMA,
        ],
    )
    def segsum(tabs, src2d, dst2d, zeros, out,
               src_all, dst_all, rows0, rows1, acc, sg0, sg1, ss0, ss1):
        cid = lax.axis_index("c")
        sid = lax.axis_index("s")
        rows_v = (rows0, rows1)
        sem_g = (sg0, sg1)
        sem_s = (ss0, ss1)
        if two_tables:
            tab = tabs.at[cid]
            cb = sid * nchunk
        else:
            tab = tabs
            cb = (cid * NS + sid) * nchunk

        r0 = pl.multiple_of(sid * ZR, 8)
        pltpu.async_copy(zeros.at[pl.ds(0, ZR)], acc.at[pl.ds(r0, ZR)],
                         sem_s[0])

        @pl.when(sid == 0)
        def _():
            pltpu.async_copy(zeros.at[pl.ds(0, NPAD - NS * ZR)],
                             acc.at[pl.ds(NS * ZR, NPAD - NS * ZR)], sem_s[1])

        for phase in range(nphase):
            ib = pl.multiple_of(cb + phase * PCH, 8)
            pltpu.sync_copy(src2d.at[pl.ds(ib, PCH)], src_all)
            pltpu.sync_copy(dst2d.at[pl.ds(ib, PCH)], dst_all)

            pltpu.async_copy(tab.at[src_all.at[0]], rows_v[0], sem_g[0])

            if phase == 0:
                pltpu.make_async_copy(
                    zeros.at[pl.ds(0, ZR)], acc.at[pl.ds(r0, ZR)],
                    sem_s[0]).wait()

                @pl.when(sid == 0)
                def _():
                    pltpu.make_async_copy(
                        zeros.at[pl.ds(0, NPAD - NS * ZR)],
                        acc.at[pl.ds(NS * ZR, NPAD - NS * ZR)],
                        sem_s[1]).wait()

                plsc.subcore_barrier()

            def pair(j2, carry):
                for b in (0, 1):
                    j = j2 * 2 + b
                    nb = 1 - b

                    @pl.when(j >= 1)
                    def _():
                        pltpu.make_async_copy(
                            rows_v[nb], acc.at[dst_all.at[j - 1]],
                            sem_s[nb]).wait()

                    @pl.when(j + 1 < PCH)
                    def _():
                        pltpu.async_copy(
                            tab.at[src_all.at[j + 1]], rows_v[nb], sem_g[nb])

                    pltpu.make_async_copy(
                        tab.at[src_all.at[j]], rows_v[b], sem_g[b]).wait()
                    pltpu.async_copy(rows_v[b], acc.at[dst_all.at[j]],
                                     sem_s[b], add=True)
                return carry

            lax.fori_loop(0, PCH // 2, pair, 0)
            pltpu.make_async_copy(
                rows_v[1], acc.at[dst_all.at[PCH - 1]], sem_s[1]).wait()
        plsc.subcore_barrier()
        pltpu.sync_copy(acc.at[pl.ds(r0, ZR)], out.at[cid, pl.ds(r0, ZR)])

        @pl.when(sid == 0)
        def _():
            pltpu.sync_copy(acc.at[pl.ds(NS * ZR, ZTAIL)],
                            out.at[cid, pl.ds(NS * ZR, ZTAIL)])

    return segsum


_segsum = _make_segsum(False)
_segsum2 = _make_segsum(True)


BN = 2000


def _mlp(z, W1, b1, W2, b2):
    t = jnp.maximum(
        jnp.dot(z, W1, preferred_element_type=jnp.float32) + b1, 0.0
    )
    return jnp.dot(t, W2, preferred_element_type=jnp.float32) + b2


def _tc1_body(x_ref, a_ref, W10, b10, W20, b20, W11, b11, W21, b21,
              W12, b12, W22, b22, out1_ref, h23_ref):
    z = x_ref[...] + a_ref[0] + a_ref[1]
    out1_ref[...] = _mlp(z, W10[...], b10[...], W20[...], b20[...])
    h23_ref[0] = _mlp(z, W11[...], b11[...], W21[...], b21[...])
    h23_ref[1] = _mlp(z, W12[...], b12[...], W22[...], b22[...])


def _tc2_body(h23_ref, a23_ref, W11, b11, W21, b21,
              W12, b12, W22, b22, out2_ref, h3b_ref):
    z2 = h23_ref[0] + a23_ref[0]
    out2_ref[...] = _mlp(z2, W11[...], b11[...], W21[...], b21[...])
    z3 = h23_ref[1] + a23_ref[1]
    h3b_ref[...] = _mlp(z3, W12[...], b12[...], W22[...], b22[...])


def _tc3_body(h3b_ref, a4_ref, out1_ref, out2_ref, W12, b12, W22, b22,
              Wp_ref, bp_ref, y_ref):
    z = h3b_ref[...] + a4_ref[0] + a4_ref[1]
    out3 = _mlp(z, W12[...], b12[...], W22[...], b22[...])
    Wp = Wp_ref[...]
    y = jnp.dot(out1_ref[...], Wp[0:H], preferred_element_type=jnp.float32)
    y += jnp.dot(out2_ref[...], Wp[H:2 * H], preferred_element_type=jnp.float32)
    y += jnp.dot(out3, Wp[2 * H:3 * H], preferred_element_type=jnp.float32)
    y_ref[...] = y + bp_ref[...]


_row_spec = pl.BlockSpec((BN, H), lambda i: (i, 0))
_agg_spec = pl.BlockSpec((NC, BN, H), lambda i: (0, i, 0))
_w_spec = pl.BlockSpec((H, H), lambda i: (0, 0))
_b_spec = pl.BlockSpec((1, H), lambda i: (0, 0))
_out_nh = jax.ShapeDtypeStruct((N, H), jnp.float32)
_grid = (N // BN,)
_tc_params = pltpu.CompilerParams(dimension_semantics=("arbitrary",))

_tc1 = pl.pallas_call(
    _tc1_body,
    grid=_grid,
    in_specs=[_row_spec, _agg_spec] + [_w_spec, _b_spec] * 6,
    out_specs=[_row_spec, _agg_spec],
    out_shape=[_out_nh, jax.ShapeDtypeStruct((NC, N, H), jnp.float32)],
    compiler_params=_tc_params,
)

_tc2 = pl.pallas_call(
    _tc2_body,
    grid=_grid,
    in_specs=[_agg_spec, _agg_spec] + [_w_spec, _b_spec] * 4,
    out_specs=[_row_spec] * 2,
    out_shape=[_out_nh] * 2,
    compiler_params=_tc_params,
)

_tc3 = pl.pallas_call(
    _tc3_body,
    grid=_grid,
    in_specs=[_row_spec, _agg_spec, _row_spec, _row_spec]
    + [_w_spec, _b_spec] * 2
    + [pl.BlockSpec((3 * H, H), lambda i: (0, 0)), _b_spec],
    out_specs=_row_spec,
    out_shape=_out_nh,
    compiler_params=_tc_params,
)


def kernel(x, edge_index, W1_0, b1_0, W2_0, b2_0, W1_1, b1_1, W2_1, b2_1,
           W1_2, b1_2, W2_2, b2_2, Wp, bp):
    pad = EP - E
    src = jnp.concatenate(
        [edge_index[0], jnp.arange(pad, dtype=edge_index.dtype) % N]
    ).reshape(EP // CH, CH)
    dst = jnp.concatenate(
        [edge_index[1],
         N + (jnp.arange(pad, dtype=edge_index.dtype) % 8)]
    ).reshape(EP // CH, CH)
    zeros = jnp.zeros((ZR, H), jnp.float32)
    b1_0r, b2_0r = b1_0.reshape(1, H), b2_0.reshape(1, H)
    b1_1r, b2_1r = b1_1.reshape(1, H), b2_1.reshape(1, H)
    b1_2r, b2_2r = b1_2.reshape(1, H), b2_2.reshape(1, H)
    bpr = bp.reshape(1, H)

    agg0 = _segsum(x, src, dst, zeros)
    out1, h23 = _tc1(x, agg0, W1_0, b1_0r, W2_0, b2_0r,
                     W1_1, b1_1r, W2_1, b2_1r, W1_2, b1_2r, W2_2, b2_2r)
    agg23 = _segsum2(h23, src, dst, zeros)
    out2, h3b = _tc2(h23, agg23,
                     W1_1, b1_1r, W2_1, b2_1r, W1_2, b1_2r, W2_2, b2_2r)
    agg3b = _segsum(h3b, src, dst, zeros)
    y = _tc3(h3b, agg3b, out1, out2, W1_2, b1_2r, W2_2, b2_2r, Wp, bpr)
    return y


# speedup vs baseline: 1.0695x; 1.0000x over previous
import functools

import jax
import jax.numpy as jnp
from jax import lax
from jax.experimental import pallas as pl
from jax.experimental.pallas import tpu as pltpu
from jax.experimental.pallas import tpu_sc as plsc

N = 10000
E = 320000
H = 128

NC = 2
NS = 16
NT = NC * NS
CH = 128
NCHUNK = 80
NPHASE = 2
PCH = NCHUNK // NPHASE
ET = CH * NCHUNK
EP = NT * ET
NPAD = N + 8
ZR = 624
ZTAIL = N - NS * ZR

_mesh = plsc.VectorSubcoreMesh(
    core_axis_name="c", subcore_axis_name="s", num_cores=NC, num_subcores=NS
)


def _make_segsum(two_tables):
    nchunk = NCHUNK * NC if two_tables else NCHUNK
    nphase = NPHASE * NC if two_tables else NPHASE

    @functools.partial(
        pl.kernel,
        out_type=jax.ShapeDtypeStruct((NC, N, H), jnp.float32),
        mesh=_mesh,
        scratch_types=[
            pltpu.VMEM((PCH, CH), jnp.int32),
            pltpu.VMEM((PCH, CH), jnp.int32),
            pltpu.VMEM((CH, H), jnp.float32),
            pltpu.VMEM((CH, H), jnp.float32),
            pltpu.VMEM_SHARED((NPAD, H), jnp.float32),
            pltpu.SemaphoreType.DMA,
            pltpu.SemaphoreType.DMA,
            pltpu.SemaphoreType.DMA,
            pltpu.SemaphoreType.DMA,
        ],
    )
    def segsum(tabs, src2d, dst2d, zeros, out,
               src_all, dst_all, rows0, rows1, acc, sg0, sg1, ss0, ss1):
        cid = lax.axis_index("c")
        sid = lax.axis_index("s")
        rows_v = (rows0, rows1)
        sem_g = (sg0, sg1)
        sem_s = (ss0, ss1)
        if two_tables:
            tab = tabs.at[cid]
            cb = sid * nchunk
        else:
            tab = tabs
            cb = (cid * NS + sid) * nchunk

        r0 = pl.multiple_of(sid * ZR, 8)
        pltpu.async_copy(zeros.at[pl.ds(0, ZR)], acc.at[pl.ds(r0, ZR)],
                         sem_s[0])

        @pl.when(sid == 0)
        def _():
            pltpu.async_copy(zeros.at[pl.ds(0, NPAD - NS * ZR)],
                             acc.at[pl.ds(NS * ZR, NPAD - NS * ZR)], sem_s[1])

        for phase in range(nphase):
            ib = pl.multiple_of(cb + phase * PCH, 8)
            pltpu.sync_copy(src2d.at[pl.ds(ib, PCH)], src_all)
            pltpu.sync_copy(dst2d.at[pl.ds(ib, PCH)], dst_all)

            pltpu.async_copy(tab.at[src_all.at[0]], rows_v[0], sem_g[0])

            if phase == 0:
                pltpu.make_async_copy(
                    zeros.at[pl.ds(0, ZR)], acc.at[pl.ds(r0, ZR)],
                    sem_s[0]).wait()

                @pl.when(sid == 0)
                def _():
                    pltpu.make_async_copy(
                        zeros.at[pl.ds(0, NPAD - NS * ZR)],
                        acc.at[pl.ds(NS * ZR, NPAD - NS * ZR)],
                        sem_s[1]).wait()

                plsc.subcore_barrier()

            def pair(j2, carry):
                for b in (0, 1):
                    j = j2 * 2 + b
                    nb = 1 - b

                    @pl.when(j >= 1)
                    def _():
                        pltpu.make_async_copy(
                            rows_v[nb], acc.at[dst_all.at[j - 1]],
                            sem_s[nb]).wait()

                    @pl.when(j + 1 < PCH)
                    def _():
                        pltpu.async_copy(
                            tab.at[src_all.at[j + 1]], rows_v[nb], sem_g[nb])

                    pltpu.make_async_copy(
                        tab.at[src_all.at[j]], rows_v[b], sem_g[b]).wait()
                    pltpu.async_copy(rows_v[b], acc.at[dst_all.at[j]],
                                     sem_s[b], add=True)
                return carry

            lax.fori_loop(0, PCH // 2, pair, 0)
            pltpu.make_async_copy(
                rows_v[1], acc.at[dst_all.at[PCH - 1]], sem_s[1]).wait()
        plsc.subcore_barrier()
        pltpu.sync_copy(acc.at[pl.ds(r0, ZR)], out.at[cid, pl.ds(r0, ZR)])

        @pl.when(sid == 0)
        def _():
            pltpu.sync_copy(acc.at[pl.ds(NS * ZR, ZTAIL)],
                            out.at[cid, pl.ds(NS * ZR, ZTAIL)])

    return segsum


_segsum = _make_segsum(False)
_segsum2 = _make_segsum(True)


BN = 5000


def _mlp(z, W1, b1, W2, b2):
    t = jnp.maximum(
        jnp.dot(z, W1, preferred_element_type=jnp.float32) + b1, 0.0
    )
    return jnp.dot(t, W2, preferred_element_type=jnp.float32) + b2


def _tc1_body(x_ref, a_ref, W10, b10, W20, b20, W11, b11, W21, b21,
              W12, b12, W22, b22, out1_ref, h23_ref):
    z = x_ref[...] + a_ref[0] + a_ref[1]
    out1_ref[...] = _mlp(z, W10[...], b10[...], W20[...], b20[...])
    h23_ref[0] = _mlp(z, W11[...], b11[...], W21[...], b21[...])
    h23_ref[1] = _mlp(z, W12[...], b12[...], W22[...], b22[...])


def _tc2_body(h23_ref, a23_ref, W11, b11, W21, b21,
              W12, b12, W22, b22, out2_ref, h3b_ref):
    z2 = h23_ref[0] + a23_ref[0]
    out2_ref[...] = _mlp(z2, W11[...], b11[...], W21[...], b21[...])
    z3 = h23_ref[1] + a23_ref[1]
    h3b_ref[...] = _mlp(z3, W12[...], b12[...], W22[...], b22[...])


def _tc3_body(h3b_ref, a4_ref, out1_ref, out2_ref, W12, b12, W22, b22,
              Wp_ref, bp_ref, y_ref):
    z = h3b_ref[...] + a4_ref[0] + a4_ref[1]
    out3 = _mlp(z, W12[...], b12[...], W22[...], b22[...])
    Wp = Wp_ref[...]
    y = jnp.dot(out1_ref[...], Wp[0:H], preferred_element_type=jnp.float32)
    y += jnp.dot(out2_ref[...], Wp[H:2 * H], preferred_element_type=jnp.float32)
    y += jnp.dot(out3, Wp[2 * H:3 * H], preferred_element_type=jnp.float32)
    y_ref[...] = y + bp_ref[...]


_row_spec = pl.BlockSpec((BN, H), lambda i: (i, 0))
_agg_spec = pl.BlockSpec((NC, BN, H), lambda i: (0, i, 0))
_w_spec = pl.BlockSpec((H, H), lambda i: (0, 0))
_b_spec = pl.BlockSpec((1, H), lambda i: (0, 0))
_out_nh = jax.ShapeDtypeStruct((N, H), jnp.float32)
_grid = (N // BN,)
_tc_params = pltpu.CompilerParams(dimension_semantics=("arbitrary",))

_tc1 = pl.pallas_call(
    _tc1_body,
    grid=_grid,
    in_specs=[_row_spec, _agg_spec] + [_w_spec, _b_spec] * 6,
    out_specs=[_row_spec, _agg_spec],
    out_shape=[_out_nh, jax.ShapeDtypeStruct((NC, N, H), jnp.float32)],
    compiler_params=_tc_params,
)

_tc2 = pl.pallas_call(
    _tc2_body,
    grid=_grid,
    in_specs=[_agg_spec, _agg_spec] + [_w_spec, _b_spec] * 4,
    out_specs=[_row_spec] * 2,
    out_shape=[_out_nh] * 2,
    compiler_params=_tc_params,
)

_tc3 = pl.pallas_call(
    _tc3_body,
    grid=_grid,
    in_specs=[_row_spec, _agg_spec, _row_spec, _row_spec]
    + [_w_spec, _b_spec] * 2
    + [pl.BlockSpec((3 * H, H), lambda i: (0, 0)), _b_spec],
    out_specs=_row_spec,
    out_shape=_out_nh,
    compiler_params=_tc_params,
)


def kernel(x, edge_index, W1_0, b1_0, W2_0, b2_0, W1_1, b1_1, W2_1, b2_1,
           W1_2, b1_2, W2_2, b2_2, Wp, bp):
    pad = EP - E
    src = jnp.concatenate(
        [edge_index[0], jnp.arange(pad, dtype=edge_index.dtype) % N]
    ).reshape(EP // CH, CH)
    dst = jnp.concatenate(
        [edge_index[1],
         N + (jnp.arange(pad, dtype=edge_index.dtype) % 8)]
    ).reshape(EP // CH, CH)
    zeros = jnp.zeros((ZR, H), jnp.float32)
    b1_0r, b2_0r = b1_0.reshape(1, H), b2_0.reshape(1, H)
    b1_1r, b2_1r = b1_1.reshape(1, H), b2_1.reshape(1, H)
    b1_2r, b2_2r = b1_2.reshape(1, H), b2_2.reshape(1, H)
    bpr = bp.reshape(1, H)

    agg0 = _segsum(x, src, dst, zeros)
    out1, h23 = _tc1(x, agg0, W1_0, b1_0r, W2_0, b2_0r,
                     W1_1, b1_1r, W2_1, b2_1r, W1_2, b1_2r, W2_2, b2_2r)
    agg23 = _segsum2(h23, src, dst, zeros)
    out2, h3b = _tc2(h23, agg23,
                     W1_1, b1_1r, W2_1, b2_1r, W1_2, b1_2r, W2_2, b2_2r)
    agg3b = _segsum(h3b, src, dst, zeros)
    y = _tc3(h3b, agg3b, out1, out2, W1_2, b1_2r, W2_2, b2_2r, Wp, bpr)
    return y
